# baseline (speedup 1.0000x reference)
import functools

import jax
import jax.numpy as jnp
from jax.experimental import pallas as pl
from jax.experimental.pallas import tpu as pltpu

_TOP_K = 2
_BT = 512
_NBUF = 6


def _block_compute(xi, w, b2d, logits_v, probs_v, idx_v, i):
    dn = (((1,), (1,)), ((), ()))
    logits = (
        jax.lax.dot_general(xi, w, dimension_numbers=dn,
                            preferred_element_type=jnp.float32)
        + b2d
    )
    logits_v[pl.ds(i * _BT, _BT), :] = logits

    cols = jax.lax.broadcasted_iota(jnp.int32, logits.shape, 1)
    max1 = jnp.max(logits, axis=1, keepdims=True)
    idx1 = jnp.argmax(logits, axis=1)
    masked = jnp.where(cols == idx1[:, None], -jnp.inf, logits)
    max2 = jnp.max(masked, axis=1, keepdims=True)
    idx2 = jnp.argmax(masked, axis=1)

    e2 = jnp.exp(max2 - max1)
    denom = 1.0 + e2
    probs_v[pl.ds(i * _BT, _BT), :] = jnp.concatenate(
        [1.0 / denom, e2 / denom], axis=1)
    idx_v[pl.ds(i * _BT, _BT), :] = jnp.stack(
        [idx1, idx2], axis=1).astype(jnp.int32)


def _router_kernel(x_hbm, w_ref, b_ref, logits_hbm, probs_hbm, idx_hbm,
                   xbuf, logits_v, probs_v, idx_v, in_sems, out_sem):
    n_tokens = x_hbm.shape[0]
    nblk = n_tokens // _BT
    w = w_ref[...]
    b2d = b_ref[...]

    def in_copy(i, buf):
        return pltpu.make_async_copy(
            x_hbm.at[pl.ds(i * _BT, _BT), :], xbuf.at[buf], in_sems.at[buf])

    def out_copy(i):
        return pltpu.make_async_copy(
            logits_v.at[pl.ds(i * _BT, _BT), :],
            logits_hbm.at[pl.ds(i * _BT, _BT), :], out_sem)

    for i in range(_NBUF):
        in_copy(i, i).start()

    for i in range(nblk):
        buf = i % _NBUF
        in_copy(i, buf).wait()
        _block_compute(xbuf[buf], w, b2d, logits_v, probs_v, idx_v, i)
        out_copy(i).start()
        nxt = i + _NBUF
        if nxt < nblk:
            in_copy(nxt, buf).start()

    for i in range(nblk):
        out_copy(i).wait()
    pltpu.make_async_copy(probs_v, probs_hbm, out_sem).start()
    pltpu.make_async_copy(idx_v, idx_hbm, out_sem).start()
    pltpu.make_async_copy(probs_v, probs_hbm, out_sem).wait()
    pltpu.make_async_copy(idx_v, idx_hbm, out_sem).wait()


@jax.jit
def _run(x, w, b2d):
    n_tokens, d_model = x.shape
    n_experts = w.shape[0]
    return pl.pallas_call(
        _router_kernel,
        in_specs=[
            pl.BlockSpec(memory_space=pl.ANY),
            pl.BlockSpec(memory_space=pltpu.VMEM),
            pl.BlockSpec(memory_space=pltpu.VMEM),
        ],
        out_specs=[
            pl.BlockSpec(memory_space=pl.ANY),
            pl.BlockSpec(memory_space=pl.ANY),
            pl.BlockSpec(memory_space=pl.ANY),
        ],
        out_shape=[
            jax.ShapeDtypeStruct((n_tokens, n_experts), jnp.float32),
            jax.ShapeDtypeStruct((n_tokens, _TOP_K), jnp.float32),
            jax.ShapeDtypeStruct((n_tokens, _TOP_K), jnp.int32),
        ],
        scratch_shapes=[
            pltpu.VMEM((_NBUF, _BT, d_model), jnp.float32),
            pltpu.VMEM((n_tokens, n_experts), jnp.float32),
            pltpu.VMEM((n_tokens, _TOP_K), jnp.float32),
            pltpu.VMEM((n_tokens, _TOP_K), jnp.int32),
            pltpu.SemaphoreType.DMA((_NBUF,)),
            pltpu.SemaphoreType.DMA,
        ],
    )(x, w, b2d)


def kernel(x, W, b):
    logits, probs, idx = _run(x, W, b.reshape(1, -1))
    return (probs, idx, logits)


# speedup vs baseline: 1.0866x; 1.0866x over previous
import functools

import jax
import jax.numpy as jnp
from jax.experimental import pallas as pl
from jax.experimental.pallas import tpu as pltpu

_TOP_K = 2


def _router_kernel(x_ref, w_ref, b_ref, logits_ref, probs_ref, idx_ref):
    logits = (
        jax.lax.dot_general(
            x_ref[...], w_ref[...],
            dimension_numbers=(((1,), (1,)), ((), ())),
            preferred_element_type=jnp.float32)
        + b_ref[...]
    )
    logits_ref[...] = logits

    cols = jax.lax.broadcasted_iota(jnp.int32, logits.shape, 1)
    max1 = jnp.max(logits, axis=1, keepdims=True)
    idx1 = jnp.argmax(logits, axis=1)
    masked = jnp.where(cols == idx1[:, None], -jnp.inf, logits)
    max2 = jnp.max(masked, axis=1, keepdims=True)
    idx2 = jnp.argmax(masked, axis=1)

    e2 = jnp.exp(max2 - max1)
    denom = 1.0 + e2
    probs_ref[...] = jnp.concatenate([1.0 / denom, e2 / denom], axis=1)
    idx_ref[...] = jnp.stack([idx1, idx2], axis=1).astype(jnp.int32)


@functools.partial(jax.jit, static_argnames=("block_t",))
def _run(x, w, b2d, block_t):
    n_tokens, d_model = x.shape
    n_experts = w.shape[0]
    grid = (n_tokens // block_t,)
    return pl.pallas_call(
        _router_kernel,
        grid=grid,
        compiler_params=pltpu.CompilerParams(
            dimension_semantics=("parallel",)),
        in_specs=[
            pl.BlockSpec((block_t, d_model), lambda i: (i, 0)),
            pl.BlockSpec((n_experts, d_model), lambda i: (0, 0)),
            pl.BlockSpec((1, n_experts), lambda i: (0, 0)),
        ],
        out_specs=[
            pl.BlockSpec((block_t, n_experts), lambda i: (i, 0)),
            pl.BlockSpec((block_t, _TOP_K), lambda i: (i, 0)),
            pl.BlockSpec((block_t, _TOP_K), lambda i: (i, 0)),
        ],
        out_shape=[
            jax.ShapeDtypeStruct((n_tokens, n_experts), jnp.float32),
            jax.ShapeDtypeStruct((n_tokens, _TOP_K), jnp.float32),
            jax.ShapeDtypeStruct((n_tokens, _TOP_K), jnp.int32),
        ],
    )(x, w, b2d)


def kernel(x, W, b):
    logits, probs, idx = _run(x, W, b.reshape(1, -1), 2048)
    return (probs, idx, logits)
